# baseline (speedup 1.0000x reference)
import functools

import jax
import jax.numpy as jnp
from jax import lax
from jax.experimental import pallas as pl
from jax.experimental.pallas import tpu as pltpu
from jax.experimental.pallas import tpu_sc as plsc

DM = 768
DFF = 3072
NE = 8
TOPK = 2
BLK = 256

NW = 32
LANES = 16


def _sc_mesh():
    return plsc.VectorSubcoreMesh(core_axis_name="c", subcore_axis_name="s")


@functools.cache
def _linear_copy_kernel(n_rows):
    rows_w = n_rows // NW

    @functools.partial(
        pl.kernel,
        out_type=jax.ShapeDtypeStruct((n_rows, DM), jnp.float32),
        mesh=_sc_mesh(),
        scratch_types=[pltpu.VMEM((rows_w, DM), jnp.float32)],
    )
    def k(x_hbm, out_hbm, buf):
        wid = lax.axis_index("s") * 2 + lax.axis_index("c")
        base = wid * rows_w
        pltpu.sync_copy(x_hbm.at[pl.ds(base, rows_w)], buf)
        pltpu.sync_copy(buf, out_hbm.at[pl.ds(base, rows_w)])

    return k


@functools.cache
def _gather_rows_kernel(n_rows, n_tok):
    rows_w = n_rows // NW
    ch = 64
    nch = rows_w // ch
    assert ch * nch == rows_w and nch >= 2

    @functools.partial(
        pl.kernel,
        out_type=jax.ShapeDtypeStruct((n_rows, DM), jnp.float32),
        mesh=_sc_mesh(),
        scratch_types=[
            [pltpu.VMEM((ch,), jnp.int32) for _ in range(nch)],
            pltpu.VMEM((ch, DM), jnp.float32),
            pltpu.VMEM((ch, DM), jnp.float32),
            pltpu.SemaphoreType.DMA,
            pltpu.SemaphoreType.DMA,
        ],
    )
    def k(x_hbm, idx_hbm, out_hbm, idx_vs, rows_a, rows_b, sem_a, sem_b):
        wid = lax.axis_index("s") * 2 + lax.axis_index("c")
        base = wid * rows_w
        bufs = (rows_a, rows_b)
        sems = (sem_a, sem_b)
        for c in range(nch):
            pltpu.sync_copy(idx_hbm.at[pl.ds(base + c * ch, ch)], idx_vs[c])
        copies = [None, None]
        for c in range(nch):
            copies[c % 2] = pltpu.async_copy(
                x_hbm.at[idx_vs[c]], bufs[c % 2], sems[c % 2]
            )
            if c >= 1:
                copies[(c - 1) % 2].wait()
                pltpu.sync_copy(
                    bufs[(c - 1) % 2], out_hbm.at[pl.ds(base + (c - 1) * ch, ch)]
                )
        copies[(nch - 1) % 2].wait()
        pltpu.sync_copy(
            bufs[(nch - 1) % 2], out_hbm.at[pl.ds(base + (nch - 1) * ch, ch)]
        )

    return k


@functools.cache
def _combine_kernel(n_tok, n_rows):
    tok_w = n_tok // NW

    @functools.partial(
        pl.kernel,
        out_type=jax.ShapeDtypeStruct((n_tok, DM), jnp.float32),
        mesh=_sc_mesh(),
        scratch_types=[
            pltpu.VMEM((tok_w,), jnp.int32),
            pltpu.VMEM((tok_w,), jnp.int32),
            pltpu.VMEM((tok_w, DM), jnp.float32),
            pltpu.VMEM((tok_w, DM), jnp.float32),
            pltpu.SemaphoreType.DMA,
        ],
    )
    def k(y_hbm, ia_hbm, ib_hbm, out_hbm, ia_v, ib_v, ra_v, rb_v, sem):
        wid = lax.axis_index("s") * 2 + lax.axis_index("c")
        base = wid * tok_w
        pltpu.sync_copy(ia_hbm.at[pl.ds(base, tok_w)], ia_v)
        pltpu.sync_copy(ib_hbm.at[pl.ds(base, tok_w)], ib_v)
        ca = pltpu.async_copy(y_hbm.at[ia_v], ra_v, sem)
        cb = pltpu.async_copy(y_hbm.at[ib_v], rb_v, sem)
        ca.wait()
        cb.wait()

        def body(i, carry):
            for j in range(DM // LANES):
                sl = pl.ds(j * LANES, LANES)
                ra_v[i, sl] = ra_v[i, sl] + rb_v[i, sl]
            return carry

        lax.fori_loop(0, tok_w, body, 0)
        pltpu.sync_copy(ra_v, out_hbm.at[pl.ds(base, tok_w)])

    return k


def _ffn_body(be_ref, xs_ref, w1_ref, b1_ref, w2_ref, b2_ref, g_ref, out_ref):
    xb = xs_ref[...]
    h = jnp.dot(xb, w1_ref[0], preferred_element_type=jnp.float32) + b1_ref[0]
    h = jnp.maximum(h, 0.0)
    y = jnp.dot(h, w2_ref[0], preferred_element_type=jnp.float32) + b2_ref[0]
    out_ref[...] = y * g_ref[0, 0][:, None]


def _grouped_ffn(nblk, x_s, w1, b1, w2, b2, gate_s, blk_expert):
    p = nblk * BLK
    grid_spec = pltpu.PrefetchScalarGridSpec(
        num_scalar_prefetch=1,
        grid=(nblk,),
        in_specs=[
            pl.BlockSpec((BLK, DM), lambda i, be: (i, 0)),
            pl.BlockSpec((1, DM, DFF), lambda i, be: (be[i], 0, 0)),
            pl.BlockSpec((1, 1, DFF), lambda i, be: (be[i], 0, 0)),
            pl.BlockSpec((1, DFF, DM), lambda i, be: (be[i], 0, 0)),
            pl.BlockSpec((1, 1, DM), lambda i, be: (be[i], 0, 0)),
            pl.BlockSpec((1, 1, BLK), lambda i, be: (i, 0, 0)),
        ],
        out_specs=pl.BlockSpec((BLK, DM), lambda i, be: (i, 0)),
    )
    return pl.pallas_call(
        _ffn_body,
        grid_spec=grid_spec,
        out_shape=jax.ShapeDtypeStruct((p, DM), jnp.float32),
    )(
        blk_expert,
        x_s,
        w1,
        b1.reshape(NE, 1, DFF),
        w2,
        b2.reshape(NE, 1, DM),
        gate_s.reshape(nblk, 1, BLK),
    )


def kernel(x, gate_w, gate_b, w1, b1, w2, b2):
    B, S, D = x.shape
    T = B * S
    AT = T * TOPK
    P = AT + NE * BLK
    nblk = P // BLK
    x_flat = x.reshape(T, D)

    logits = x_flat @ gate_w + gate_b
    gates = jax.nn.softmax(logits, axis=-1)
    topk_vals, topk_idx = jax.lax.top_k(gates, TOPK)

    e_flat = topk_idx.reshape(AT).astype(jnp.int32)
    oh = (e_flat[:, None] == jnp.arange(NE, dtype=jnp.int32)[None, :]).astype(
        jnp.int32
    )
    csum = jnp.cumsum(oh, axis=0)
    rank = jnp.take_along_axis(csum, e_flat[:, None], axis=1)[:, 0] - 1
    counts = csum[-1]
    padded = ((counts + BLK - 1) // BLK) * BLK
    pad_off = jnp.concatenate(
        [jnp.zeros((1,), jnp.int32), jnp.cumsum(padded)[:-1].astype(jnp.int32)]
    )
    dest = pad_off[e_flat] + rank
    tok_of_assign = jnp.arange(AT, dtype=jnp.int32) // TOPK
    src_token = jnp.zeros((P,), jnp.int32).at[dest].set(tok_of_assign)
    gate_s = jnp.zeros((P,), jnp.float32).at[dest].set(topk_vals.reshape(AT))
    blk_expert = (
        jnp.searchsorted(
            (pad_off // BLK).astype(jnp.int32),
            jnp.arange(nblk, dtype=jnp.int32),
            side="right",
        ).astype(jnp.int32)
        - 1
    )
    blk_expert = jnp.clip(blk_expert, 0, NE - 1)
    dest2 = dest.reshape(T, TOPK)

    x_lin = _linear_copy_kernel(T)(x_flat)
    x_s = _gather_rows_kernel(P, T)(x_lin, src_token)
    y_s = _grouped_ffn(nblk, x_s, w1, b1, w2, b2, gate_s, blk_expert)
    out_flat = _combine_kernel(T, P)(y_s, dest2[:, 0], dest2[:, 1])
    return out_flat.reshape(B, S, D)


# speedup vs baseline: 1.6018x; 1.6018x over previous
import functools

import jax
import jax.numpy as jnp
from jax import lax
from jax.experimental import pallas as pl
from jax.experimental.pallas import tpu as pltpu
from jax.experimental.pallas import tpu_sc as plsc

DM = 768
DFF = 3072
NE = 8
TOPK = 2
BLK = 256

NW = 32
LANES = 16


def _sc_mesh():
    return plsc.VectorSubcoreMesh(core_axis_name="c", subcore_axis_name="s")


@functools.cache
def _scatter_dispatch_kernel(n_tok, n_rows):
    tok_w = n_tok // NW

    @functools.partial(
        pl.kernel,
        out_type=jax.ShapeDtypeStruct((n_rows, DM), jnp.float32),
        mesh=_sc_mesh(),
        scratch_types=[
            pltpu.VMEM((tok_w,), jnp.int32),
            pltpu.VMEM((tok_w,), jnp.int32),
            pltpu.VMEM((tok_w, DM), jnp.float32),
            pltpu.SemaphoreType.DMA,
            pltpu.SemaphoreType.DMA,
        ],
    )
    def k(x_hbm, ia_hbm, ib_hbm, out_hbm, ia_v, ib_v, rows_v, sem_a, sem_b):
        wid = lax.axis_index("s") * 2 + lax.axis_index("c")
        base = wid * tok_w
        pltpu.sync_copy(ia_hbm.at[pl.ds(base, tok_w)], ia_v)
        pltpu.sync_copy(ib_hbm.at[pl.ds(base, tok_w)], ib_v)
        pltpu.sync_copy(x_hbm.at[pl.ds(base, tok_w)], rows_v)
        ca = pltpu.async_copy(rows_v, out_hbm.at[ia_v], sem_a)
        cb = pltpu.async_copy(rows_v, out_hbm.at[ib_v], sem_b)
        ca.wait()
        cb.wait()

    return k


@functools.cache
def _combine_kernel(n_tok, n_rows):
    tok_w = n_tok // NW

    @functools.partial(
        pl.kernel,
        out_type=jax.ShapeDtypeStruct((n_tok, DM), jnp.float32),
        mesh=_sc_mesh(),
        scratch_types=[
            pltpu.VMEM((tok_w,), jnp.int32),
            pltpu.VMEM((tok_w,), jnp.int32),
            pltpu.VMEM((tok_w, DM), jnp.float32),
            pltpu.VMEM((tok_w, DM), jnp.float32),
            pltpu.SemaphoreType.DMA,
        ],
    )
    def k(y_hbm, ia_hbm, ib_hbm, out_hbm, ia_v, ib_v, ra_v, rb_v, sem):
        wid = lax.axis_index("s") * 2 + lax.axis_index("c")
        base = wid * tok_w
        pltpu.sync_copy(ia_hbm.at[pl.ds(base, tok_w)], ia_v)
        pltpu.sync_copy(ib_hbm.at[pl.ds(base, tok_w)], ib_v)
        ca = pltpu.async_copy(y_hbm.at[ia_v], ra_v, sem)
        cb = pltpu.async_copy(y_hbm.at[ib_v], rb_v, sem)
        ca.wait()
        cb.wait()

        def body(i, carry):
            for j in range(DM // LANES):
                sl = pl.ds(j * LANES, LANES)
                ra_v[i, sl] = ra_v[i, sl] + rb_v[i, sl]
            return carry

        lax.fori_loop(0, tok_w, body, 0)
        pltpu.sync_copy(ra_v, out_hbm.at[pl.ds(base, tok_w)])

    return k


def _ffn_body(be_ref, xs_ref, w1_ref, b1_ref, w2_ref, b2_ref, g_ref, out_ref):
    xb = xs_ref[...]
    h = jnp.dot(xb, w1_ref[0], preferred_element_type=jnp.float32) + b1_ref[0]
    h = jnp.maximum(h, 0.0)
    y = jnp.dot(h, w2_ref[0], preferred_element_type=jnp.float32) + b2_ref[0]
    out_ref[...] = y * g_ref[0, 0][:, None]


def _grouped_ffn(nblk, x_s, w1, b1, w2, b2, gate_s, blk_expert):
    p = nblk * BLK
    grid_spec = pltpu.PrefetchScalarGridSpec(
        num_scalar_prefetch=1,
        grid=(nblk,),
        in_specs=[
            pl.BlockSpec((BLK, DM), lambda i, be: (i, 0)),
            pl.BlockSpec((1, DM, DFF), lambda i, be: (be[i], 0, 0)),
            pl.BlockSpec((1, 1, DFF), lambda i, be: (be[i], 0, 0)),
            pl.BlockSpec((1, DFF, DM), lambda i, be: (be[i], 0, 0)),
            pl.BlockSpec((1, 1, DM), lambda i, be: (be[i], 0, 0)),
            pl.BlockSpec((1, 1, BLK), lambda i, be: (i, 0, 0)),
        ],
        out_specs=pl.BlockSpec((BLK, DM), lambda i, be: (i, 0)),
    )
    return pl.pallas_call(
        _ffn_body,
        grid_spec=grid_spec,
        out_shape=jax.ShapeDtypeStruct((p, DM), jnp.float32),
    )(
        blk_expert,
        x_s,
        w1,
        b1.reshape(NE, 1, DFF),
        w2,
        b2.reshape(NE, 1, DM),
        gate_s.reshape(nblk, 1, BLK),
    )


def kernel(x, gate_w, gate_b, w1, b1, w2, b2):
    B, S, D = x.shape
    T = B * S
    AT = T * TOPK
    P = AT + NE * BLK
    nblk = P // BLK
    x_flat = x.reshape(T, D)

    logits = x_flat @ gate_w + gate_b
    gates = jax.nn.softmax(logits, axis=-1)
    topk_vals, topk_idx = jax.lax.top_k(gates, TOPK)

    e_flat = topk_idx.reshape(AT).astype(jnp.int32)
    oh = (e_flat[:, None] == jnp.arange(NE, dtype=jnp.int32)[None, :]).astype(
        jnp.int32
    )
    csum = jnp.cumsum(oh, axis=0)
    rank = jnp.take_along_axis(csum, e_flat[:, None], axis=1)[:, 0] - 1
    counts = csum[-1]
    padded = ((counts + BLK - 1) // BLK) * BLK
    pad_off = jnp.concatenate(
        [jnp.zeros((1,), jnp.int32), jnp.cumsum(padded)[:-1].astype(jnp.int32)]
    )
    dest = pad_off[e_flat] + rank
    gate_s = jnp.zeros((P,), jnp.float32).at[dest].set(topk_vals.reshape(AT))
    blk_expert = (
        jnp.searchsorted(
            (pad_off // BLK).astype(jnp.int32),
            jnp.arange(nblk, dtype=jnp.int32),
            side="right",
        ).astype(jnp.int32)
        - 1
    )
    blk_expert = jnp.clip(blk_expert, 0, NE - 1)
    dest2 = dest.reshape(T, TOPK)

    x_s = _scatter_dispatch_kernel(T, P)(x_flat, dest2[:, 0], dest2[:, 1])
    y_s = _grouped_ffn(nblk, x_s, w1, b1, w2, b2, gate_s, blk_expert)
    out_flat = _combine_kernel(T, P)(y_s, dest2[:, 0], dest2[:, 1])
    return out_flat.reshape(B, S, D)
